# baseline (speedup 1.0000x reference)
import functools
import math

import jax
import jax.numpy as jnp
from jax import lax
from jax.experimental import pallas as pl
from jax.experimental.pallas import tpu as pltpu
from jax.experimental.pallas import tpu_sc as plsc

_V = 100000
_B = 1024
_SMOOTH = 0.1
_CONF = 1.0 - _SMOOTH
_EPS = _SMOOTH / (_V - 1)
_CONST = _B * ((_V - 1) * _EPS * math.log(_EPS) + _CONF * math.log(_CONF))

_NW = 32
_RPW = _B // _NW
_L = 16

_RTC = 0
_SC_ROWS = _B - _RTC
_SRPW = _SC_ROWS // _NW
_CW = 4096
_NCC = _V // _CW
_TAILW = 1792
_TAILVEC = (_V - _NCC * _CW) // _L

_CROWS = 8
_NCHUNK = _RTC // _CROWS
_NBUF = 8


def _sum_body(x_hbm, o_ref, buf, sems):
    def start(c, b):
        pltpu.make_async_copy(
            x_hbm.at[pl.ds(c * _CROWS, _CROWS), :], buf.at[b], sems.at[b]
        ).start()

    def wait(c, b):
        pltpu.make_async_copy(
            x_hbm.at[pl.ds(c * _CROWS, _CROWS), :], buf.at[b], sems.at[b]
        ).wait()

    for b in range(_NBUF):
        start(b, b)

    def outer(g, acc):
        for b in range(_NBUF):
            c = g * _NBUF + b
            wait(c, b)
            acc = acc + jnp.sum(buf[b])

            @pl.when(c + _NBUF < _NCHUNK)
            def _():
                start(c + _NBUF, b)

        return acc

    o_ref[0, 0] = lax.fori_loop(
        0, _NCHUNK // _NBUF, outer, jnp.float32(0.0), unroll=False
    )


def _combine_body(g_ref, s_ref, o_ref):
    o_ref[0, 0] = (jnp.float32(_CONST)
                   - jnp.float32(_EPS) * jnp.sum(s_ref[...])
                   - jnp.float32(_CONF - _EPS) * jnp.sum(g_ref[...]))


def _combine(gparts, sparts):
    out = pl.pallas_call(
        _combine_body,
        out_specs=pl.BlockSpec(memory_space=pltpu.SMEM),
        out_shape=jax.ShapeDtypeStruct((1, 1), jnp.float32),
    )(gparts, sparts)
    return out[0, 0]


def _sc_gather_body(x_hbm, tgt_hbm, out_hbm, tgt_v, gath_v, acc_v, gsem):
    wid = lax.axis_index("s") * 2 + lax.axis_index("c")

    gbase = wid * _RPW
    pltpu.sync_copy(tgt_hbm.at[pl.ds(gbase, _RPW)], tgt_v)
    gcopies = []
    for grp in range(_RPW // _L):
        tv = tgt_v[pl.ds(grp * _L, _L)]
        for jj in range(_L):
            j = grp * _L + jj
            t = tv[jj]
            ct0 = pl.multiple_of(lax.bitwise_and(t, jnp.int32(~127)), 128)
            cp = pltpu.make_async_copy(
                x_hbm.at[pl.ds(gbase + (j // 8) * 8, 8), pl.ds(ct0, 128)],
                gath_v.at[j],
                gsem,
            )
            cp.start()
            gcopies.append(cp)
    for cp in gcopies:
        cp.wait()
    gacc = jnp.zeros((_L,), jnp.float32)
    lanes = lax.iota(jnp.int32, _L)
    for grp in range(_RPW // _L):
        tv = tgt_v[pl.ds(grp * _L, _L)]
        lanev = lax.bitwise_and(tv, jnp.int32(_L - 1))
        c0v = lax.bitwise_and(tv, jnp.int32(112))
        for jj in range(_L):
            j = grp * _L + jj
            vec = gath_v[j, j % 8, pl.ds(c0v[jj], _L)]
            gacc = gacc + jnp.where(lanes == lanev[jj], vec, jnp.float32(0.0))
    acc_v[...] = gacc
    pltpu.sync_copy(acc_v, out_hbm.at[wid])


def _sc_sum_body(x_hbm, out_hbm, buf_v, acc_v, ssem):
    wid = lax.axis_index("s") * 2 + lax.axis_index("c")

    def chunk_copy(rb, c, b, width):
        rB = pl.multiple_of(_RTC + wid * _SRPW + rb * 8, 8)
        co = pl.multiple_of(
            jnp.int32(c * _CW) + wid * jnp.int32(0), _CW if width == _CW else 128
        )
        return pltpu.make_async_copy(
            x_hbm.at[pl.ds(rB, 8), pl.ds(co, width)],
            buf_v.at[b, :, pl.ds(0, width)],
            ssem,
        )

    accs = [jnp.zeros((_L,), jnp.float32) for _ in range(4)]

    def reduce_full(b, accs):
        a0, a1, a2, a3 = accs
        for r in range(8):
            def body(i, carry):
                c0, c1, c2, c3 = carry
                base = i * 64
                c0 = c0 + buf_v[b, r, pl.ds(base, _L)]
                c1 = c1 + buf_v[b, r, pl.ds(base + 16, _L)]
                c2 = c2 + buf_v[b, r, pl.ds(base + 32, _L)]
                c3 = c3 + buf_v[b, r, pl.ds(base + 48, _L)]
                return c0, c1, c2, c3
            a0, a1, a2, a3 = lax.fori_loop(0, _CW // 64, body, (a0, a1, a2, a3))
        return [a0, a1, a2, a3]

    def reduce_tail(b, accs):
        a0, a1, a2, a3 = accs
        for r in range(8):
            def body(i, carry):
                c0, c1 = carry
                base = i * 32
                c0 = c0 + buf_v[b, r, pl.ds(base, _L)]
                c1 = c1 + buf_v[b, r, pl.ds(base + 16, _L)]
                return c0, c1
            a0, a1 = lax.fori_loop(0, _TAILVEC // 2, body, (a0, a1))
        return [a0, a1, a2, a3]

    def band(rb, accs):
        accs = list(accs)
        chunk_copy(rb, 0, 0, _CW).start()
        for c in range(_NCC + 1):
            if c + 1 <= _NCC:
                nb = (c + 1) % 2
                if c + 1 < _NCC:
                    chunk_copy(rb, c + 1, nb, _CW).start()
                else:
                    chunk_copy(rb, _NCC, nb, _TAILW).start()
            b = c % 2
            if c < _NCC:
                chunk_copy(rb, c, b, _CW).wait()
                accs = reduce_full(b, accs)
            else:
                chunk_copy(rb, _NCC, b, _TAILW).wait()
                accs = reduce_tail(b, accs)
        return tuple(accs)

    accs = lax.fori_loop(0, _SRPW // 8, band, tuple(accs))
    acc_v[...] = (accs[0] + accs[1]) + (accs[2] + accs[3])
    pltpu.sync_copy(acc_v, out_hbm.at[wid])


def _sc_gather(x, tgt):
    mesh = plsc.VectorSubcoreMesh(core_axis_name="c", subcore_axis_name="s")
    k = functools.partial(
        pl.kernel,
        mesh=mesh,
        out_type=jax.ShapeDtypeStruct((_NW, _L), jnp.float32),
        scratch_types=[
            pltpu.VMEM((_RPW,), jnp.int32),
            pltpu.VMEM((_RPW, 8, 128), jnp.float32),
            pltpu.VMEM((_L,), jnp.float32),
            pltpu.SemaphoreType.DMA,
        ],
    )(_sc_gather_body)
    return k(x, tgt)


def _sc_sum(x):
    mesh = plsc.VectorSubcoreMesh(core_axis_name="c", subcore_axis_name="s")
    k = functools.partial(
        pl.kernel,
        mesh=mesh,
        out_type=jax.ShapeDtypeStruct((_NW, _L), jnp.float32),
        scratch_types=[
            pltpu.VMEM((2, 8, _CW), jnp.float32),
            pltpu.VMEM((_L,), jnp.float32),
            pltpu.SemaphoreType.DMA,
        ],
    )(_sc_sum_body)
    return k(x)


def kernel(x, target):
    tgt = target.astype(jnp.int32)
    gparts = _sc_gather(x, tgt)
    sparts = _sc_sum(x)
    return _combine(gparts, sparts)


# speedup vs baseline: 1.1765x; 1.1765x over previous
import functools
import math

import jax
import jax.numpy as jnp
from jax import lax
from jax.experimental import pallas as pl
from jax.experimental.pallas import tpu as pltpu
from jax.experimental.pallas import tpu_sc as plsc

_V = 100000
_B = 1024
_SMOOTH = 0.1
_CONF = 1.0 - _SMOOTH
_EPS = _SMOOTH / (_V - 1)
_CONST = _B * ((_V - 1) * _EPS * math.log(_EPS) + _CONF * math.log(_CONF))

_NW = 32
_RPW = _B // _NW
_L = 16

_RTC = 0
_SC_ROWS = _B - _RTC
_SRPW = _SC_ROWS // _NW
_CW = 4096
_NCC = _V // _CW
_TAILW = 1792
_TAILVEC = (_V - _NCC * _CW) // _L

_CROWS = 8
_NCHUNK = _RTC // _CROWS
_NBUF = 8


def _sum_body(x_hbm, o_ref, buf, sems):
    def start(c, b):
        pltpu.make_async_copy(
            x_hbm.at[pl.ds(c * _CROWS, _CROWS), :], buf.at[b], sems.at[b]
        ).start()

    def wait(c, b):
        pltpu.make_async_copy(
            x_hbm.at[pl.ds(c * _CROWS, _CROWS), :], buf.at[b], sems.at[b]
        ).wait()

    for b in range(_NBUF):
        start(b, b)

    def outer(g, acc):
        for b in range(_NBUF):
            c = g * _NBUF + b
            wait(c, b)
            acc = acc + jnp.sum(buf[b])

            @pl.when(c + _NBUF < _NCHUNK)
            def _():
                start(c + _NBUF, b)

        return acc

    o_ref[0, 0] = lax.fori_loop(
        0, _NCHUNK // _NBUF, outer, jnp.float32(0.0), unroll=False
    )


_NS = 4
_SROWS = _B // _NS
_SBLK = 16


def _msum_body(x0_ref, x1_ref, x2_ref, x3_ref, o_ref):
    i = pl.program_id(0)

    @pl.when(i == 0)
    def _():
        o_ref[0, 0] = jnp.float32(0.0)

    o_ref[0, 0] += (
        (jnp.sum(x0_ref[...]) + jnp.sum(x1_ref[...]))
        + (jnp.sum(x2_ref[...]) + jnp.sum(x3_ref[...]))
    )


def _tc_msum(x):
    nblk = _SROWS // _SBLK
    specs = [
        pl.BlockSpec((_SBLK, _V), lambda i, k=k: (i + k * nblk, 0))
        for k in range(_NS)
    ]
    out = pl.pallas_call(
        _msum_body,
        grid=(nblk,),
        in_specs=specs,
        out_specs=pl.BlockSpec(memory_space=pltpu.SMEM),
        out_shape=jax.ShapeDtypeStruct((1, 1), jnp.float32),
    )(x, x, x, x)
    return out[0, 0]


def _combine_body(g_ref, s_ref, o_ref):
    o_ref[0, 0] = (jnp.float32(_CONST)
                   - jnp.float32(_EPS) * jnp.sum(s_ref[...])
                   - jnp.float32(_CONF - _EPS) * jnp.sum(g_ref[...]))


def _combine(gparts, sparts):
    out = pl.pallas_call(
        _combine_body,
        out_specs=pl.BlockSpec(memory_space=pltpu.SMEM),
        out_shape=jax.ShapeDtypeStruct((1, 1), jnp.float32),
    )(gparts, sparts)
    return out[0, 0]


def _sc_gather_body(x_hbm, tgt_hbm, out_hbm, tgt_v, gath_v, acc_v, gsem):
    wid = lax.axis_index("s") * 2 + lax.axis_index("c")

    gbase = wid * _RPW
    pltpu.sync_copy(tgt_hbm.at[pl.ds(gbase, _RPW)], tgt_v)
    gcopies = []
    for grp in range(_RPW // _L):
        tv = tgt_v[pl.ds(grp * _L, _L)]
        for jj in range(_L):
            j = grp * _L + jj
            t = tv[jj]
            ct0 = pl.multiple_of(lax.bitwise_and(t, jnp.int32(~127)), 128)
            cp = pltpu.make_async_copy(
                x_hbm.at[pl.ds(gbase + (j // 8) * 8, 8), pl.ds(ct0, 128)],
                gath_v.at[j],
                gsem,
            )
            cp.start()
            gcopies.append(cp)
    for cp in gcopies:
        cp.wait()
    gacc = jnp.zeros((_L,), jnp.float32)
    lanes = lax.iota(jnp.int32, _L)
    for grp in range(_RPW // _L):
        tv = tgt_v[pl.ds(grp * _L, _L)]
        lanev = lax.bitwise_and(tv, jnp.int32(_L - 1))
        c0v = lax.bitwise_and(tv, jnp.int32(112))
        for jj in range(_L):
            j = grp * _L + jj
            vec = gath_v[j, j % 8, pl.ds(c0v[jj], _L)]
            gacc = gacc + jnp.where(lanes == lanev[jj], vec, jnp.float32(0.0))
    acc_v[...] = gacc
    pltpu.sync_copy(acc_v, out_hbm.at[wid])


def _sc_sum_body(x_hbm, out_hbm, buf_v, acc_v, ssem):
    wid = lax.axis_index("s") * 2 + lax.axis_index("c")

    def chunk_copy(rb, c, b, width):
        rB = pl.multiple_of(_RTC + wid * _SRPW + rb * 8, 8)
        co = pl.multiple_of(
            jnp.int32(c * _CW) + wid * jnp.int32(0), _CW if width == _CW else 128
        )
        return pltpu.make_async_copy(
            x_hbm.at[pl.ds(rB, 8), pl.ds(co, width)],
            buf_v.at[b, :, pl.ds(0, width)],
            ssem,
        )

    accs = [jnp.zeros((_L,), jnp.float32) for _ in range(4)]

    def reduce_full(b, accs):
        a0, a1, a2, a3 = accs
        for r in range(8):
            def body(i, carry):
                c0, c1, c2, c3 = carry
                base = i * 64
                c0 = c0 + buf_v[b, r, pl.ds(base, _L)]
                c1 = c1 + buf_v[b, r, pl.ds(base + 16, _L)]
                c2 = c2 + buf_v[b, r, pl.ds(base + 32, _L)]
                c3 = c3 + buf_v[b, r, pl.ds(base + 48, _L)]
                return c0, c1, c2, c3
            a0, a1, a2, a3 = lax.fori_loop(0, _CW // 64, body, (a0, a1, a2, a3))
        return [a0, a1, a2, a3]

    def reduce_tail(b, accs):
        a0, a1, a2, a3 = accs
        for r in range(8):
            def body(i, carry):
                c0, c1 = carry
                base = i * 32
                c0 = c0 + buf_v[b, r, pl.ds(base, _L)]
                c1 = c1 + buf_v[b, r, pl.ds(base + 16, _L)]
                return c0, c1
            a0, a1 = lax.fori_loop(0, _TAILVEC // 2, body, (a0, a1))
        return [a0, a1, a2, a3]

    def band(rb, accs):
        accs = list(accs)
        chunk_copy(rb, 0, 0, _CW).start()
        for c in range(_NCC + 1):
            if c + 1 <= _NCC:
                nb = (c + 1) % 2
                if c + 1 < _NCC:
                    chunk_copy(rb, c + 1, nb, _CW).start()
                else:
                    chunk_copy(rb, _NCC, nb, _TAILW).start()
            b = c % 2
            if c < _NCC:
                chunk_copy(rb, c, b, _CW).wait()
                accs = reduce_full(b, accs)
            else:
                chunk_copy(rb, _NCC, b, _TAILW).wait()
                accs = reduce_tail(b, accs)
        return tuple(accs)

    accs = lax.fori_loop(0, _SRPW // 8, band, tuple(accs))
    acc_v[...] = (accs[0] + accs[1]) + (accs[2] + accs[3])
    pltpu.sync_copy(acc_v, out_hbm.at[wid])


def _sc_gather(x, tgt):
    mesh = plsc.VectorSubcoreMesh(core_axis_name="c", subcore_axis_name="s")
    k = functools.partial(
        pl.kernel,
        mesh=mesh,
        out_type=jax.ShapeDtypeStruct((_NW, _L), jnp.float32),
        scratch_types=[
            pltpu.VMEM((_RPW,), jnp.int32),
            pltpu.VMEM((_RPW, 8, 128), jnp.float32),
            pltpu.VMEM((_L,), jnp.float32),
            pltpu.SemaphoreType.DMA,
        ],
    )(_sc_gather_body)
    return k(x, tgt)


def _sc_sum(x):
    mesh = plsc.VectorSubcoreMesh(core_axis_name="c", subcore_axis_name="s")
    k = functools.partial(
        pl.kernel,
        mesh=mesh,
        out_type=jax.ShapeDtypeStruct((_NW, _L), jnp.float32),
        scratch_types=[
            pltpu.VMEM((2, 8, _CW), jnp.float32),
            pltpu.VMEM((_L,), jnp.float32),
            pltpu.SemaphoreType.DMA,
        ],
    )(_sc_sum_body)
    return k(x)


def kernel(x, target):
    tgt = target.astype(jnp.int32)
    gparts = _sc_gather(x, tgt)
    s = _tc_msum(x)
    g = jnp.sum(gparts)
    return (jnp.float32(_CONST) - jnp.float32(_EPS) * s
            - jnp.float32(_CONF - _EPS) * g)


# speedup vs baseline: 1.1907x; 1.0121x over previous
import functools
import math

import jax
import jax.numpy as jnp
from jax import lax
from jax.experimental import pallas as pl
from jax.experimental.pallas import tpu as pltpu
from jax.experimental.pallas import tpu_sc as plsc

_V = 100000
_B = 1024
_SMOOTH = 0.1
_CONF = 1.0 - _SMOOTH
_EPS = _SMOOTH / (_V - 1)
_CONST = _B * ((_V - 1) * _EPS * math.log(_EPS) + _CONF * math.log(_CONF))

_NW = 32
_RPW = _B // _NW
_L = 16

_RTC = 0
_SC_ROWS = _B - _RTC
_SRPW = _SC_ROWS // _NW
_CW = 4096
_NCC = _V // _CW
_TAILW = 1792
_TAILVEC = (_V - _NCC * _CW) // _L

_CROWS = 8
_NCHUNK = _B // _CROWS
_NBUF = 8


def _sum_body(x_hbm, o_ref, buf, sems):
    def start(c, b):
        pltpu.make_async_copy(
            x_hbm.at[pl.ds(c * _CROWS, _CROWS), :], buf.at[b], sems.at[b]
        ).start(priority=b % 2)

    def wait(c, b):
        pltpu.make_async_copy(
            x_hbm.at[pl.ds(c * _CROWS, _CROWS), :], buf.at[b], sems.at[b]
        ).wait()

    for b in range(_NBUF):
        start(b, b)

    def outer(g, acc):
        for b in range(_NBUF):
            c = g * _NBUF + b
            wait(c, b)
            acc = acc + jnp.sum(buf[b])

            @pl.when(c + _NBUF < _NCHUNK)
            def _():
                start(c + _NBUF, b)

        return acc

    o_ref[0, 0] = lax.fori_loop(
        0, _NCHUNK // _NBUF, outer, jnp.float32(0.0), unroll=False
    )


_NS = 4
_SROWS = _B // _NS
_SBLK = 16


def _msum_body(x0_ref, x1_ref, x2_ref, x3_ref, o_ref):
    i = pl.program_id(0)

    @pl.when(i == 0)
    def _():
        o_ref[0, 0] = jnp.float32(0.0)

    o_ref[0, 0] += (
        (jnp.sum(x0_ref[...]) + jnp.sum(x1_ref[...]))
        + (jnp.sum(x2_ref[...]) + jnp.sum(x3_ref[...]))
    )


def _tc_msum(x):
    nblk = _SROWS // _SBLK
    specs = [
        pl.BlockSpec((_SBLK, _V), lambda i, k=k: (i + k * nblk, 0))
        for k in range(_NS)
    ]
    out = pl.pallas_call(
        _msum_body,
        grid=(nblk,),
        in_specs=specs,
        out_specs=pl.BlockSpec(memory_space=pltpu.SMEM),
        out_shape=jax.ShapeDtypeStruct((1, 1), jnp.float32),
    )(x, x, x, x)
    return out[0, 0]


def _tc_ring_sum(x):
    out = pl.pallas_call(
        _sum_body,
        in_specs=[pl.BlockSpec(memory_space=pl.ANY)],
        out_specs=pl.BlockSpec(memory_space=pltpu.SMEM),
        out_shape=jax.ShapeDtypeStruct((1, 1), jnp.float32),
        scratch_shapes=[
            pltpu.VMEM((_NBUF, _CROWS, _V), jnp.float32),
            pltpu.SemaphoreType.DMA((_NBUF,)),
        ],
    )(x)
    return out[0, 0]


def _combine_body(g_ref, s_ref, o_ref):
    o_ref[0, 0] = (jnp.float32(_CONST)
                   - jnp.float32(_EPS) * jnp.sum(s_ref[...])
                   - jnp.float32(_CONF - _EPS) * jnp.sum(g_ref[...]))


def _combine(gparts, sparts):
    out = pl.pallas_call(
        _combine_body,
        out_specs=pl.BlockSpec(memory_space=pltpu.SMEM),
        out_shape=jax.ShapeDtypeStruct((1, 1), jnp.float32),
    )(gparts, sparts)
    return out[0, 0]


def _sc_gather_body(x_hbm, tgt_hbm, out_hbm, tgt_v, gath_v, acc_v, gsem):
    wid = lax.axis_index("s") * 2 + lax.axis_index("c")

    gbase = wid * _RPW
    pltpu.sync_copy(tgt_hbm.at[pl.ds(gbase, _RPW)], tgt_v)
    gcopies = []
    for grp in range(_RPW // _L):
        tv = tgt_v[pl.ds(grp * _L, _L)]
        for jj in range(_L):
            j = grp * _L + jj
            t = tv[jj]
            ct0 = pl.multiple_of(lax.bitwise_and(t, jnp.int32(~127)), 128)
            cp = pltpu.make_async_copy(
                x_hbm.at[pl.ds(gbase + (j // 8) * 8, 8), pl.ds(ct0, 128)],
                gath_v.at[j],
                gsem,
            )
            cp.start()
            gcopies.append(cp)
    for cp in gcopies:
        cp.wait()
    gacc = jnp.zeros((_L,), jnp.float32)
    lanes = lax.iota(jnp.int32, _L)
    for grp in range(_RPW // _L):
        tv = tgt_v[pl.ds(grp * _L, _L)]
        lanev = lax.bitwise_and(tv, jnp.int32(_L - 1))
        c0v = lax.bitwise_and(tv, jnp.int32(112))
        for jj in range(_L):
            j = grp * _L + jj
            vec = gath_v[j, j % 8, pl.ds(c0v[jj], _L)]
            gacc = gacc + jnp.where(lanes == lanev[jj], vec, jnp.float32(0.0))
    acc_v[...] = gacc
    pltpu.sync_copy(acc_v, out_hbm.at[wid])


def _sc_sum_body(x_hbm, out_hbm, buf_v, acc_v, ssem):
    wid = lax.axis_index("s") * 2 + lax.axis_index("c")

    def chunk_copy(rb, c, b, width):
        rB = pl.multiple_of(_RTC + wid * _SRPW + rb * 8, 8)
        co = pl.multiple_of(
            jnp.int32(c * _CW) + wid * jnp.int32(0), _CW if width == _CW else 128
        )
        return pltpu.make_async_copy(
            x_hbm.at[pl.ds(rB, 8), pl.ds(co, width)],
            buf_v.at[b, :, pl.ds(0, width)],
            ssem,
        )

    accs = [jnp.zeros((_L,), jnp.float32) for _ in range(4)]

    def reduce_full(b, accs):
        a0, a1, a2, a3 = accs
        for r in range(8):
            def body(i, carry):
                c0, c1, c2, c3 = carry
                base = i * 64
                c0 = c0 + buf_v[b, r, pl.ds(base, _L)]
                c1 = c1 + buf_v[b, r, pl.ds(base + 16, _L)]
                c2 = c2 + buf_v[b, r, pl.ds(base + 32, _L)]
                c3 = c3 + buf_v[b, r, pl.ds(base + 48, _L)]
                return c0, c1, c2, c3
            a0, a1, a2, a3 = lax.fori_loop(0, _CW // 64, body, (a0, a1, a2, a3))
        return [a0, a1, a2, a3]

    def reduce_tail(b, accs):
        a0, a1, a2, a3 = accs
        for r in range(8):
            def body(i, carry):
                c0, c1 = carry
                base = i * 32
                c0 = c0 + buf_v[b, r, pl.ds(base, _L)]
                c1 = c1 + buf_v[b, r, pl.ds(base + 16, _L)]
                return c0, c1
            a0, a1 = lax.fori_loop(0, _TAILVEC // 2, body, (a0, a1))
        return [a0, a1, a2, a3]

    def band(rb, accs):
        accs = list(accs)
        chunk_copy(rb, 0, 0, _CW).start()
        for c in range(_NCC + 1):
            if c + 1 <= _NCC:
                nb = (c + 1) % 2
                if c + 1 < _NCC:
                    chunk_copy(rb, c + 1, nb, _CW).start()
                else:
                    chunk_copy(rb, _NCC, nb, _TAILW).start()
            b = c % 2
            if c < _NCC:
                chunk_copy(rb, c, b, _CW).wait()
                accs = reduce_full(b, accs)
            else:
                chunk_copy(rb, _NCC, b, _TAILW).wait()
                accs = reduce_tail(b, accs)
        return tuple(accs)

    accs = lax.fori_loop(0, _SRPW // 8, band, tuple(accs))
    acc_v[...] = (accs[0] + accs[1]) + (accs[2] + accs[3])
    pltpu.sync_copy(acc_v, out_hbm.at[wid])


def _sc_gather(x, tgt):
    mesh = plsc.VectorSubcoreMesh(core_axis_name="c", subcore_axis_name="s")
    k = functools.partial(
        pl.kernel,
        mesh=mesh,
        out_type=jax.ShapeDtypeStruct((_NW, _L), jnp.float32),
        scratch_types=[
            pltpu.VMEM((_RPW,), jnp.int32),
            pltpu.VMEM((_RPW, 8, 128), jnp.float32),
            pltpu.VMEM((_L,), jnp.float32),
            pltpu.SemaphoreType.DMA,
        ],
    )(_sc_gather_body)
    return k(x, tgt)


def _sc_sum(x):
    mesh = plsc.VectorSubcoreMesh(core_axis_name="c", subcore_axis_name="s")
    k = functools.partial(
        pl.kernel,
        mesh=mesh,
        out_type=jax.ShapeDtypeStruct((_NW, _L), jnp.float32),
        scratch_types=[
            pltpu.VMEM((2, 8, _CW), jnp.float32),
            pltpu.VMEM((_L,), jnp.float32),
            pltpu.SemaphoreType.DMA,
        ],
    )(_sc_sum_body)
    return k(x)


def kernel(x, target):
    tgt = target.astype(jnp.int32)
    gparts = _sc_gather(x, tgt)
    s = _tc_ring_sum(x)
    g = jnp.sum(gparts)
    return (jnp.float32(_CONST) - jnp.float32(_EPS) * s
            - jnp.float32(_CONF - _EPS) * g)
